# baseline (speedup 1.0000x reference)
import jax
import jax.numpy as jnp
from jax import lax
from jax.experimental import pallas as pl
from jax.experimental.pallas import tpu as pltpu
from jax.experimental.pallas import tpu_sc as plsc

_MAX_DEGREE = 512
_HIDDEN = 128
_N = 100000

_NC = 2
_NS = 16
_NW = _NC * _NS

_CHUNK = 112
_FULL = 3136
_LAST = 2784
_TAIL_BASE = 31 * _FULL + _LAST - 96
_TAIL = _N - _TAIL_BASE


def _body(deg_hbm, table_hbm, out_hbm,
          bounce, table_sh, idxa, wb0, wb1, gs0, gs1, ws0, ws1, ts):
    c = lax.axis_index("c")
    s = lax.axis_index("s")
    wid = s * _NC + c
    base = wid * _FULL
    last = wid == _NW - 1

    rows_per = _MAX_DEGREE // _NS
    pltpu.sync_copy(table_hbm.at[pl.ds(s * rows_per, rows_per)], bounce)
    pltpu.sync_copy(bounce, table_sh.at[pl.ds(s * rows_per, rows_per)])

    @pl.when(jnp.logical_not(last))
    def _():
        pltpu.sync_copy(deg_hbm.at[pl.ds(base, _FULL)], idxa)

    @pl.when(last)
    def _():
        pltpu.sync_copy(deg_hbm.at[pl.ds(base, _LAST)], idxa.at[pl.ds(0, _LAST)])
        pltpu.sync_copy(deg_hbm.at[pl.ds(_TAIL_BASE, _TAIL)],
                        idxa.at[pl.ds(_LAST, _TAIL)])

    def clamp(lo, hi):
        for i in range(lo, hi):
            sl = pl.ds(i * 16, 16)
            idxa[sl] = jnp.minimum(jnp.maximum(idxa[sl], 0), _MAX_DEGREE - 1)

    clamp(0, (_LAST + _TAIL) // 16)

    @pl.when(jnp.logical_not(last))
    def _():
        clamp((_LAST + _TAIL) // 16, _FULL // 16)

    def fire_gather(idx_off, wb, sem):
        return pltpu.async_copy(
            table_sh.at[idxa.at[pl.ds(idx_off, _CHUNK)]], wb, sem)

    def fire_gather_hbm(idx_off, wb, sem):
        return pltpu.async_copy(
            table_hbm.at[idxa.at[pl.ds(idx_off, _CHUNK)]], wb, sem)

    def fire_write(row_off, wb, sem):
        return pltpu.async_copy(
            wb, out_hbm.at[pl.ds(row_off, _CHUNK)], sem)

    def drain_write(sem):
        pltpu.make_async_copy(
            wb0, out_hbm.at[pl.ds(0, _CHUNK)], sem).wait()

    g0 = fire_gather_hbm(0, wb0, gs0)
    g1 = fire_gather_hbm(_CHUNK, wb1, gs1)
    g0.wait()
    fire_write(base, wb0, ws0)
    g1.wait()
    fire_write(base + _CHUNK, wb1, ws1)

    plsc.subcore_barrier()

    n_pairs = jnp.where(last, 12, 14)

    @pl.loop(1, n_pairs)
    def _(t):
        off = t * 2 * _CHUNK
        drain_write(ws0)
        fire_gather(off, wb0, gs0).wait()
        fire_write(base + off, wb0, ws0)
        drain_write(ws1)

        @pl.when(t % 3 == 0)
        def _():
            fire_gather_hbm(off + _CHUNK, wb1, gs1).wait()

        @pl.when(t % 3 != 0)
        def _():
            fire_gather(off + _CHUNK, wb1, gs1).wait()

        fire_write(base + off + _CHUNK, wb1, ws1)

    @pl.when(jnp.logical_not(last))
    def _():
        drain_write(ws0)
        drain_write(ws1)

    @pl.when(last)
    def _():
        drain_write(ws0)
        pltpu.async_copy(
            table_sh.at[idxa.at[pl.ds(_LAST, _TAIL)]],
            wb0.at[pl.ds(0, _TAIL)], gs0).wait()
        pltpu.async_copy(
            wb0.at[pl.ds(0, _TAIL)],
            out_hbm.at[pl.ds(_TAIL_BASE, _TAIL)], ts).wait()
        drain_write(ws1)


@jax.jit
def _run(degrees, table):
    mesh = plsc.VectorSubcoreMesh(core_axis_name="c", subcore_axis_name="s")
    k = pl.kernel(
        _body,
        mesh=mesh,
        compiler_params=pltpu.CompilerParams(needs_layout_passes=False),
        out_type=jax.ShapeDtypeStruct((_N, _HIDDEN), jnp.float32),
        scratch_types=[
            pltpu.VMEM((_MAX_DEGREE // _NS, _HIDDEN), jnp.float32),
            pltpu.VMEM_SHARED((_MAX_DEGREE, _HIDDEN), jnp.float32),
            pltpu.VMEM((_FULL,), jnp.int32),
            pltpu.VMEM((_CHUNK, _HIDDEN), jnp.float32),
            pltpu.VMEM((_CHUNK, _HIDDEN), jnp.float32),
            pltpu.SemaphoreType.DMA,
            pltpu.SemaphoreType.DMA,
            pltpu.SemaphoreType.DMA,
            pltpu.SemaphoreType.DMA,
            pltpu.SemaphoreType.DMA,
        ],
    )
    return k(degrees, table)


def kernel(degrees, degree_embedding):
    return _run(degrees.astype(jnp.int32), degree_embedding)


# speedup vs baseline: 1.1685x; 1.1685x over previous
import jax
import jax.numpy as jnp
from jax import lax
from jax.experimental import pallas as pl
from jax.experimental.pallas import tpu as pltpu
from jax.experimental.pallas import tpu_sc as plsc

_MAX_DEGREE = 512
_HIDDEN = 128
_N = 100000

_NC = 2
_NS = 16
_NW = _NC * _NS

_CHUNK = 112
_FULL = 3136
_LAST = 2784
_TAIL_BASE = 31 * _FULL + _LAST - 96
_TAIL = _N - _TAIL_BASE


def _body(deg_hbm, table_hbm, out_hbm,
          bounce, table_sh, idxa, wb0, wb1, gs0, gs1, ws0, ws1, ts):
    c = lax.axis_index("c")
    s = lax.axis_index("s")
    wid = s * _NC + c
    base = wid * _FULL
    last = wid == _NW - 1

    rows_per = _MAX_DEGREE // _NS
    pltpu.sync_copy(table_hbm.at[pl.ds(s * rows_per, rows_per)], bounce)
    pltpu.sync_copy(bounce, table_sh.at[pl.ds(s * rows_per, rows_per)])

    @pl.when(jnp.logical_not(last))
    def _():
        pltpu.sync_copy(deg_hbm.at[pl.ds(base, _FULL)], idxa)

    @pl.when(last)
    def _():
        pltpu.sync_copy(deg_hbm.at[pl.ds(base, _LAST)], idxa.at[pl.ds(0, _LAST)])
        pltpu.sync_copy(deg_hbm.at[pl.ds(_TAIL_BASE, _TAIL)],
                        idxa.at[pl.ds(_LAST, _TAIL)])

    def clamp(lo, hi):
        for i in range(lo, hi):
            sl = pl.ds(i * 16, 16)
            idxa[sl] = jnp.minimum(jnp.maximum(idxa[sl], 0), _MAX_DEGREE - 1)

    clamp(0, (_LAST + _TAIL) // 16)

    @pl.when(jnp.logical_not(last))
    def _():
        clamp((_LAST + _TAIL) // 16, _FULL // 16)

    def fire_gather(idx_off, wb, sem):
        return pltpu.async_copy(
            table_sh.at[idxa.at[pl.ds(idx_off, _CHUNK)]], wb, sem)

    def fire_gather_hbm(idx_off, wb, sem):
        return pltpu.async_copy(
            table_hbm.at[idxa.at[pl.ds(idx_off, _CHUNK)]], wb, sem)

    def fire_write(row_off, wb, sem):
        return pltpu.async_copy(
            wb, out_hbm.at[pl.ds(row_off, _CHUNK)], sem)

    def drain_write(sem):
        pltpu.make_async_copy(
            wb0, out_hbm.at[pl.ds(0, _CHUNK)], sem).wait()

    g0 = fire_gather_hbm(0, wb0, gs0)
    g1 = fire_gather_hbm(_CHUNK, wb1, gs1)
    g0.wait()
    fire_write(base, wb0, ws0)
    g1.wait()
    fire_write(base + _CHUNK, wb1, ws1)

    plsc.subcore_barrier()

    n_pairs = jnp.where(last, 12, 14)

    @pl.loop(1, n_pairs)
    def _(t):
        off = t * 2 * _CHUNK
        drain_write(ws0)
        fire_gather(off, wb0, gs0).wait()
        fire_write(base + off, wb0, ws0)
        drain_write(ws1)
        fire_gather(off + _CHUNK, wb1, gs1).wait()
        fire_write(base + off + _CHUNK, wb1, ws1)

    @pl.when(jnp.logical_not(last))
    def _():
        drain_write(ws0)
        drain_write(ws1)

    @pl.when(last)
    def _():
        drain_write(ws0)
        pltpu.async_copy(
            table_sh.at[idxa.at[pl.ds(_LAST, _TAIL)]],
            wb0.at[pl.ds(0, _TAIL)], gs0).wait()
        pltpu.async_copy(
            wb0.at[pl.ds(0, _TAIL)],
            out_hbm.at[pl.ds(_TAIL_BASE, _TAIL)], ts).wait()
        drain_write(ws1)


@jax.jit
def _run(degrees, table):
    mesh = plsc.VectorSubcoreMesh(core_axis_name="c", subcore_axis_name="s")
    k = pl.kernel(
        _body,
        mesh=mesh,
        compiler_params=pltpu.CompilerParams(needs_layout_passes=False),
        out_type=jax.ShapeDtypeStruct((_N, _HIDDEN), jnp.float32),
        scratch_types=[
            pltpu.VMEM((_MAX_DEGREE // _NS, _HIDDEN), jnp.float32),
            pltpu.VMEM_SHARED((_MAX_DEGREE, _HIDDEN), jnp.float32),
            pltpu.VMEM((_FULL,), jnp.int32),
            pltpu.VMEM((_CHUNK, _HIDDEN), jnp.float32),
            pltpu.VMEM((_CHUNK, _HIDDEN), jnp.float32),
            pltpu.SemaphoreType.DMA,
            pltpu.SemaphoreType.DMA,
            pltpu.SemaphoreType.DMA,
            pltpu.SemaphoreType.DMA,
            pltpu.SemaphoreType.DMA,
        ],
    )
    return k(degrees, table)


def kernel(degrees, degree_embedding):
    return _run(degrees.astype(jnp.int32), degree_embedding)


# speedup vs baseline: 1.2855x; 1.1002x over previous
import jax
import jax.numpy as jnp
from jax import lax
from jax.experimental import pallas as pl
from jax.experimental.pallas import tpu as pltpu
from jax.experimental.pallas import tpu_sc as plsc

_MAX_DEGREE = 512
_HIDDEN = 128
_N = 100000

_NC = 2
_NS = 16
_NW = _NC * _NS

_CHUNK = 112
_FULL = 3136
_LAST = 2784
_TAIL_BASE = 31 * _FULL + _LAST - 96
_TAIL = _N - _TAIL_BASE


def _body(deg_hbm, table_hbm, out_hbm,
          bounce, table_sh, idxa, wb0, wb1, gs0, gs1, ws0, ws1, ts):
    c = lax.axis_index("c")
    s = lax.axis_index("s")
    wid = s * _NC + c
    base = wid * _FULL
    last = wid == _NW - 1

    rows_per = _MAX_DEGREE // _NS
    pltpu.sync_copy(table_hbm.at[pl.ds(s * rows_per, rows_per)], bounce)
    pltpu.sync_copy(bounce, table_sh.at[pl.ds(s * rows_per, rows_per)])

    @pl.when(jnp.logical_not(last))
    def _():
        pltpu.sync_copy(deg_hbm.at[pl.ds(base, _FULL)], idxa)

    @pl.when(last)
    def _():
        pltpu.sync_copy(deg_hbm.at[pl.ds(base, _LAST)], idxa.at[pl.ds(0, _LAST)])
        pltpu.sync_copy(deg_hbm.at[pl.ds(_TAIL_BASE, _TAIL)],
                        idxa.at[pl.ds(_LAST, _TAIL)])

    def clamp(lo, hi):
        for i in range(lo, hi):
            sl = pl.ds(i * 16, 16)
            idxa[sl] = jnp.minimum(jnp.maximum(idxa[sl], 0), _MAX_DEGREE - 1)

    clamp(0, (_LAST + _TAIL) // 16)

    @pl.when(jnp.logical_not(last))
    def _():
        clamp((_LAST + _TAIL) // 16, _FULL // 16)

    def fire_gather(idx_off, wb, sem):
        return pltpu.async_copy(
            table_sh.at[idxa.at[pl.ds(idx_off, _CHUNK)]], wb, sem)

    def fire_write(row_off, wb, sem):
        return pltpu.async_copy(
            wb, out_hbm.at[pl.ds(row_off, _CHUNK)], sem)

    def drain_write(sem):
        pltpu.make_async_copy(
            wb0, out_hbm.at[pl.ds(0, _CHUNK)], sem).wait()

    plsc.subcore_barrier()

    fire_gather(0, wb0, gs0).wait()
    fire_write(base, wb0, ws0)
    fire_gather(_CHUNK, wb1, gs1).wait()
    fire_write(base + _CHUNK, wb1, ws1)

    n_pairs = jnp.where(last, 12, 14)

    @pl.loop(1, n_pairs)
    def _(t):
        off = t * 2 * _CHUNK
        drain_write(ws0)
        ga = fire_gather(off, wb0, gs0)
        drain_write(ws1)
        gb = fire_gather(off + _CHUNK, wb1, gs1)
        ga.wait()
        fire_write(base + off, wb0, ws0)
        gb.wait()
        fire_write(base + off + _CHUNK, wb1, ws1)

    @pl.when(jnp.logical_not(last))
    def _():
        drain_write(ws0)
        drain_write(ws1)

    @pl.when(last)
    def _():
        drain_write(ws0)
        pltpu.async_copy(
            table_sh.at[idxa.at[pl.ds(_LAST, _TAIL)]],
            wb0.at[pl.ds(0, _TAIL)], gs0).wait()
        pltpu.async_copy(
            wb0.at[pl.ds(0, _TAIL)],
            out_hbm.at[pl.ds(_TAIL_BASE, _TAIL)], ts).wait()
        drain_write(ws1)


@jax.jit
def _run(degrees, table):
    mesh = plsc.VectorSubcoreMesh(core_axis_name="c", subcore_axis_name="s")
    k = pl.kernel(
        _body,
        mesh=mesh,
        compiler_params=pltpu.CompilerParams(needs_layout_passes=False),
        out_type=jax.ShapeDtypeStruct((_N, _HIDDEN), jnp.float32),
        scratch_types=[
            pltpu.VMEM((_MAX_DEGREE // _NS, _HIDDEN), jnp.float32),
            pltpu.VMEM_SHARED((_MAX_DEGREE, _HIDDEN), jnp.float32),
            pltpu.VMEM((_FULL,), jnp.int32),
            pltpu.VMEM((_CHUNK, _HIDDEN), jnp.float32),
            pltpu.VMEM((_CHUNK, _HIDDEN), jnp.float32),
            pltpu.SemaphoreType.DMA,
            pltpu.SemaphoreType.DMA,
            pltpu.SemaphoreType.DMA,
            pltpu.SemaphoreType.DMA,
            pltpu.SemaphoreType.DMA,
        ],
    )
    return k(degrees, table)


def kernel(degrees, degree_embedding):
    return _run(degrees.astype(jnp.int32), degree_embedding)


# speedup vs baseline: 1.2876x; 1.0016x over previous
import jax
import jax.numpy as jnp
from jax import lax
from jax.experimental import pallas as pl
from jax.experimental.pallas import tpu as pltpu
from jax.experimental.pallas import tpu_sc as plsc

_MAX_DEGREE = 512
_HIDDEN = 128
_N = 100000

_NC = 2
_NS = 16
_NW = _NC * _NS

_CHUNK = 112
_FULL = 3136
_LAST = 2784
_TAIL_BASE = 31 * _FULL + _LAST - 96
_TAIL = _N - _TAIL_BASE


def _body(deg_hbm, table_hbm, out_hbm,
          bounce, table_sh, idxa, wb0, wb1, gs0, gs1, ws0, ws1, ts):
    c = lax.axis_index("c")
    s = lax.axis_index("s")
    wid = s * _NC + c
    base = wid * _FULL
    last = wid == _NW - 1

    rows_per = _MAX_DEGREE // _NS
    pltpu.sync_copy(table_hbm.at[pl.ds(s * rows_per, rows_per)], bounce)
    pltpu.sync_copy(bounce, table_sh.at[pl.ds(s * rows_per, rows_per)])

    @pl.when(jnp.logical_not(last))
    def _():
        pltpu.sync_copy(deg_hbm.at[pl.ds(base, _FULL)], idxa)

    @pl.when(last)
    def _():
        pltpu.sync_copy(deg_hbm.at[pl.ds(base, _LAST)], idxa.at[pl.ds(0, _LAST)])
        pltpu.sync_copy(deg_hbm.at[pl.ds(_TAIL_BASE, _TAIL)],
                        idxa.at[pl.ds(_LAST, _TAIL)])

    def clamp(lo, hi):
        for i in range(lo, hi):
            sl = pl.ds(i * 16, 16)
            idxa[sl] = jnp.minimum(jnp.maximum(idxa[sl], 0), _MAX_DEGREE - 1)

    clamp(0, (_LAST + _TAIL) // 16)

    @pl.when(jnp.logical_not(last))
    def _():
        clamp((_LAST + _TAIL) // 16, _FULL // 16)

    def fire_gather(idx_off, wb, sem):
        return pltpu.async_copy(
            table_sh.at[idxa.at[pl.ds(idx_off, _CHUNK)]], wb, sem)

    def fire_write(row_off, wb, sem):
        return pltpu.async_copy(
            wb, out_hbm.at[pl.ds(row_off, _CHUNK)], sem)

    def drain_write(sem):
        pltpu.make_async_copy(
            wb0, out_hbm.at[pl.ds(0, _CHUNK)], sem).wait()

    plsc.subcore_barrier()

    fire_gather(0, wb0, gs0).wait()
    fire_write(base, wb0, ws0)
    fire_gather(_CHUNK, wb1, gs1).wait()
    fire_write(base + _CHUNK, wb1, ws1)

    n_pairs = jnp.where(last, 12, 14)

    @pl.loop(1, n_pairs)
    def _(t):
        off = t * 2 * _CHUNK
        drain_write(ws0)
        ga = fire_gather(off, wb0, gs0)
        drain_write(ws1)
        gb = fire_gather(off + _CHUNK, wb1, gs1)
        ga.wait()
        fire_write(base + off, wb0, ws0)
        gb.wait()
        fire_write(base + off + _CHUNK, wb1, ws1)

    @pl.when(jnp.logical_not(last))
    def _():
        drain_write(ws0)
        drain_write(ws1)

    @pl.when(last)
    def _():
        drain_write(ws0)
        pltpu.async_copy(
            table_sh.at[idxa.at[pl.ds(_LAST, _TAIL)]],
            wb0.at[pl.ds(0, _TAIL)], gs0).wait()
        pltpu.async_copy(
            wb0.at[pl.ds(0, _TAIL)],
            out_hbm.at[pl.ds(_TAIL_BASE, _TAIL)], ts).wait()
        drain_write(ws1)


@jax.jit
def _run(degrees, table):
    mesh = plsc.VectorSubcoreMesh(core_axis_name="c", subcore_axis_name="s")
    k = pl.kernel(
        _body,
        mesh=mesh,
        compiler_params=pltpu.CompilerParams(
            needs_layout_passes=False, skip_device_barrier=True),
        out_type=jax.ShapeDtypeStruct((_N, _HIDDEN), jnp.float32),
        scratch_types=[
            pltpu.VMEM((_MAX_DEGREE // _NS, _HIDDEN), jnp.float32),
            pltpu.VMEM_SHARED((_MAX_DEGREE, _HIDDEN), jnp.float32),
            pltpu.VMEM((_FULL,), jnp.int32),
            pltpu.VMEM((_CHUNK, _HIDDEN), jnp.float32),
            pltpu.VMEM((_CHUNK, _HIDDEN), jnp.float32),
            pltpu.SemaphoreType.DMA,
            pltpu.SemaphoreType.DMA,
            pltpu.SemaphoreType.DMA,
            pltpu.SemaphoreType.DMA,
            pltpu.SemaphoreType.DMA,
        ],
    )
    return k(degrees, table)


def kernel(degrees, degree_embedding):
    return _run(degrees.astype(jnp.int32), degree_embedding)
